# baseline (speedup 1.0000x reference)
import functools

import jax
import jax.numpy as jnp
from jax import lax
from jax.experimental import pallas as pl
from jax.experimental.pallas import tpu as pltpu
from jax.experimental.pallas import tpu_sc as plsc

_LANES = 128


def _diag_gather_sc(flat, idx, batch, chunks):
    mesh = plsc.VectorSubcoreMesh(core_axis_name="c", subcore_axis_name="s")
    num_cores = 2

    @functools.partial(
        pl.kernel,
        mesh=mesh,
        out_type=jax.ShapeDtypeStruct((batch, chunks, _LANES), jnp.float32),
        scratch_types=[
            pltpu.VMEM((chunks, _LANES), jnp.int32),
            pltpu.VMEM((chunks, _LANES), jnp.float32),
            pltpu.SemaphoreType.DMA,
        ],
    )
    def gather_kernel(flat_hbm, idx_hbm, out_hbm, idx_v, vals_v, sem):
        wid = lax.axis_index("s") * num_cores + lax.axis_index("c")
        pltpu.sync_copy(idx_hbm.at[wid], idx_v)
        copies = [
            pltpu.async_copy(flat_hbm.at[idx_v.at[j]], vals_v.at[j], sem)
            for j in range(chunks)
        ]
        for c in copies:
            c.wait()
        pltpu.sync_copy(vals_v, out_hbm.at[wid])

    return gather_kernel(flat, idx)


def _project_tc(diag, p_pad, batch, s):
    def body(d_ref, p_ref, o_ref):
        o_ref[...] = jnp.dot(
            d_ref[...], p_ref[...], preferred_element_type=jnp.float32
        )

    return pl.pallas_call(
        body,
        out_shape=jax.ShapeDtypeStruct((batch, s), jnp.float32),
    )(diag, p_pad)


def kernel(input, P):
    batch, dim, _ = input.shape
    s = P.shape[1]
    dpad = ((dim + _LANES - 1) // _LANES) * _LANES
    chunks = dpad // _LANES

    r = jnp.minimum(jnp.arange(dpad, dtype=jnp.int32), dim - 1)
    base = jnp.arange(batch, dtype=jnp.int32) * (dim * dim)
    idx = (base[:, None] + r[None, :] * (dim + 1)).reshape(batch, chunks, _LANES)

    diag = _diag_gather_sc(input.reshape(-1), idx, batch, chunks)
    p_pad = jnp.pad(P, ((0, dpad - dim), (0, 0)))
    return _project_tc(diag.reshape(batch, dpad), p_pad, batch, s)


# speedup vs baseline: 13.5660x; 13.5660x over previous
import functools

import jax
import jax.numpy as jnp
from jax import lax
from jax.experimental import pallas as pl
from jax.experimental.pallas import tpu as pltpu
from jax.experimental.pallas import tpu_sc as plsc

_SUB = 8


def _diag_gather_sc(inp, batch, dim, dpad):
    mesh = plsc.VectorSubcoreMesh(core_axis_name="c", subcore_axis_name="s")
    num_cores = 2
    nfull = dim // _SUB

    @functools.partial(
        pl.kernel,
        mesh=mesh,
        out_type=jax.ShapeDtypeStruct((batch, dpad), jnp.float32),
        scratch_types=[
            pltpu.VMEM((dpad // 128, _SUB, 128), jnp.float32),
            pltpu.VMEM((dpad,), jnp.float32),
            pltpu.SemaphoreType.DMA,
        ],
        compiler_params=pltpu.CompilerParams(
            use_tc_tiling_on_sc=True, needs_layout_passes=False
        ),
    )
    def gather_kernel(inp_hbm, out_hbm, vals_v, diag_v, sem):
        wid = lax.axis_index("s") * num_cores + lax.axis_index("c")

        for m in range(16):
            nq = (nfull - m + 15) // 16

            def issue(q, _, m=m):
                row0 = 128 * q + _SUB * m
                band = inp_hbm.at[wid, pl.ds(row0, _SUB), pl.ds(128 * q, 128)]
                pltpu.async_copy(
                    band.at[:, pl.ds(_SUB * m, _SUB)],
                    vals_v.at[q, pl.ds(0, _SUB), pl.ds(_SUB * m, _SUB)],
                    sem,
                )
                return _

            lax.fori_loop(0, nq, issue, 0, unroll=2)

        def drain(k, _):
            pltpu.make_async_copy(
                inp_hbm.at[wid, pl.ds(0, _SUB), pl.ds(0, _SUB)],
                vals_v.at[0, pl.ds(0, _SUB), pl.ds(0, _SUB)],
                sem,
            ).wait()
            return _

        lax.fori_loop(0, nfull, drain, 0, unroll=2)
        lane = lax.iota(jnp.int32, 16)
        for i in range(dpad // 16):
            g = jnp.minimum(lane + (16 * i), _SUB * nfull - 1)
            q = lax.shift_right_logical(g, 7)
            j = lax.bitwise_and(g, 7)
            c = lax.bitwise_and(g, 127)
            diag_v[pl.ds(16 * i, 16)] = plsc.load_gather(vals_v, [q, j, c])
        pltpu.sync_copy(diag_v, out_hbm.at[wid])

    return gather_kernel(inp)


def _project_tc(diag, tail, p_pad, batch, dim, dpad, s):
    rem = dim % _SUB
    base = dim - rem

    def body(d_ref, t_ref, p_ref, o_ref):
        d = d_ref[...]
        if rem:
            col = lax.broadcasted_iota(jnp.int32, (batch, dpad), 1)
            for x in range(rem):
                fix = t_ref[:, x * rem + x][:, None]
                d = jnp.where(col == base + x, fix, d)
        o_ref[...] = jnp.dot(d, p_ref[...], preferred_element_type=jnp.float32)

    return pl.pallas_call(
        body,
        out_shape=jax.ShapeDtypeStruct((batch, s), jnp.float32),
    )(diag, tail, p_pad)


def kernel(input, P):
    batch, dim, _ = input.shape
    s = P.shape[1]
    dpad = ((dim + 127) // 128) * 128
    rem = dim % _SUB
    base = dim - rem

    diag = _diag_gather_sc(input, batch, dim, dpad)
    tail = input[:, base:, base:].reshape(batch, max(rem * rem, 1))
    p_pad = jnp.pad(P, ((0, dpad - dim), (0, 0)))
    return _project_tc(diag, tail, p_pad, batch, dim, dpad, s)
